# baseline (speedup 1.0000x reference)
import functools

import jax
import jax.numpy as jnp
from jax import lax
from jax.experimental import pallas as pl
from jax.experimental.pallas import tpu as pltpu
from jax.experimental.pallas import tpu_sc as plsc
from jax.experimental.layout import Layout, with_layout_constraint

_NC, _NS = 2, 16
_NW = _NC * _NS
_WINDOW = 512
_SENT = -4


def kernel(input, weight):
    batch, hist = input.shape
    _, dim = weight.shape
    n = batch * hist

    scaled = input.reshape(n) * 4
    cols = [scaled] + [jnp.full_like(scaled, _SENT)] * 3
    idx_list = jnp.stack(cols, axis=1).reshape(-1)

    weight = with_layout_constraint(
        weight, Layout(major_to_minor=(0, 1), tiling=((32,),))
    )

    n_win = (4 * n) // _WINDOW // _NW

    mesh = plsc.VectorSubcoreMesh(core_axis_name="c", subcore_axis_name="s")

    @functools.partial(
        pl.kernel,
        mesh=mesh,
        out_type=jax.ShapeDtypeStruct((n, dim), jnp.float32),
        scratch_types=[
            pltpu.VMEM((_WINDOW,), jnp.int32),
            pltpu.VMEM((_WINDOW, dim), jnp.float32),
            pltpu.SemaphoreType.DMA,
        ],
    )
    def gather_kernel(table_hbm, l_hbm, out_hbm, idx_v, rows_v, sem):
        wid = lax.axis_index("s") * _NC + lax.axis_index("c")

        @pl.loop(0, n_win)
        def _(win):
            base = (wid * n_win + win) * _WINDOW
            pltpu.sync_copy(l_hbm.at[pl.ds(base, _WINDOW)], idx_v)
            src = table_hbm.at[plsc.Indices(idx_v, ignored_value=_SENT)]
            pltpu.async_copy(src, rows_v, sem).wait()
            base4 = pl.multiple_of(base // 4, _WINDOW // 4)
            pltpu.sync_copy(
                rows_v.at[pl.ds(0, _WINDOW // 4)],
                out_hbm.at[pl.ds(base4, _WINDOW // 4)],
            )

    out = gather_kernel(weight, idx_list)
    return out.reshape(batch, hist, dim)


# speedup vs baseline: 1.1151x; 1.1151x over previous
import functools

import jax
import jax.numpy as jnp
from jax import lax
from jax.experimental import pallas as pl
from jax.experimental.pallas import tpu as pltpu
from jax.experimental.pallas import tpu_sc as plsc
from jax.experimental.layout import Layout, with_layout_constraint

_NC, _NS = 2, 16
_NW = _NC * _NS
_SENT = -4


def kernel(input, weight):
    batch, hist = input.shape
    _, dim = weight.shape
    n = batch * hist
    window = 4 * hist

    scaled = input.reshape(n) * 4
    cols = [scaled] + [jnp.full_like(scaled, _SENT)] * 3
    idx_list = jnp.stack(cols, axis=1).reshape(-1)

    weight = with_layout_constraint(
        weight, Layout(major_to_minor=(0, 1), tiling=((32,),))
    )

    n_win = batch // _NW

    mesh = plsc.VectorSubcoreMesh(core_axis_name="c", subcore_axis_name="s")

    @functools.partial(
        pl.kernel,
        mesh=mesh,
        out_type=jax.ShapeDtypeStruct((batch, hist, dim), jnp.float32),
        scratch_types=[
            pltpu.VMEM((window,), jnp.int32),
            pltpu.VMEM((window, dim), jnp.float32),
            pltpu.SemaphoreType.DMA,
        ],
    )
    def gather_kernel(table_hbm, l_hbm, out_hbm, idx_v, rows_v, sem):
        wid = lax.axis_index("s") * _NC + lax.axis_index("c")

        @pl.loop(0, n_win)
        def _(win):
            b = wid * n_win + win
            pltpu.sync_copy(l_hbm.at[pl.ds(b * window, window)], idx_v)
            src = table_hbm.at[plsc.Indices(idx_v, ignored_value=_SENT)]
            pltpu.async_copy(src, rows_v, sem).wait()
            pltpu.sync_copy(rows_v.at[pl.ds(0, hist)], out_hbm.at[b])

    return gather_kernel(weight, idx_list)


# speedup vs baseline: 1.7194x; 1.5420x over previous
import functools

import jax
import jax.numpy as jnp
from jax import lax
from jax.experimental import pallas as pl
from jax.experimental.pallas import tpu as pltpu
from jax.experimental.pallas import tpu_sc as plsc
from jax.experimental.layout import Layout, with_layout_constraint

_NC, _NS = 2, 16
_NW = _NC * _NS
_SENT = -4


def kernel(input, weight):
    batch, hist = input.shape
    _, dim = weight.shape
    n = batch * hist
    window = 4 * hist

    scaled = input.reshape(n) * 4
    idx_list = jax.lax.pad(scaled, jnp.int32(_SENT), [(0, 3, 3)])

    weight = with_layout_constraint(
        weight, Layout(major_to_minor=(0, 1), tiling=((32,),))
    )

    n_win = batch // _NW

    mesh = plsc.VectorSubcoreMesh(core_axis_name="c", subcore_axis_name="s")

    @functools.partial(
        pl.kernel,
        mesh=mesh,
        out_type=jax.ShapeDtypeStruct((batch, hist, dim), jnp.float32),
        scratch_types=[
            pltpu.VMEM((window,), jnp.int32),
            pltpu.VMEM((window, dim), jnp.float32),
            pltpu.SemaphoreType.DMA,
        ],
    )
    def gather_kernel(table_hbm, l_hbm, out_hbm, idx_v, rows_v, sem):
        wid = lax.axis_index("s") * _NC + lax.axis_index("c")

        @pl.loop(0, n_win)
        def _(win):
            b = wid * n_win + win
            pltpu.sync_copy(l_hbm.at[pl.ds(b * window, window)], idx_v)
            src = table_hbm.at[plsc.Indices(idx_v, ignored_value=_SENT)]
            pltpu.async_copy(src, rows_v, sem).wait()
            pltpu.sync_copy(rows_v.at[pl.ds(0, hist)], out_hbm.at[b])

    return gather_kernel(weight, idx_list)


# speedup vs baseline: 2.5087x; 1.4590x over previous
import functools

import jax
import jax.numpy as jnp
from jax import lax
from jax.experimental import pallas as pl
from jax.experimental.pallas import tpu as pltpu
from jax.experimental.pallas import tpu_sc as plsc
from jax.experimental.layout import Layout, with_layout_constraint

_NC, _NS = 2, 16
_NW = _NC * _NS
_SENT = -4


def kernel(input, weight):
    batch, hist = input.shape
    _, dim = weight.shape
    n = batch * hist
    window = 4 * hist

    scaled = input.reshape(n) * 4
    idx_list = jax.lax.pad(scaled, jnp.int32(_SENT), [(0, 3, 3)])

    weight = with_layout_constraint(
        weight, Layout(major_to_minor=(0, 1), tiling=((32,),))
    )

    rows_per_win = 2
    wentries = rows_per_win * window
    n_win = batch // _NW // rows_per_win

    mesh = plsc.VectorSubcoreMesh(core_axis_name="c", subcore_axis_name="s")

    @functools.partial(
        pl.kernel,
        mesh=mesh,
        out_type=jax.ShapeDtypeStruct((batch, hist, dim), jnp.float32),
        scratch_types=[
            [pltpu.VMEM((wentries,), jnp.int32) for _ in range(2)],
            [pltpu.VMEM((wentries, dim), jnp.float32) for _ in range(2)],
            [pltpu.SemaphoreType.DMA for _ in range(2)],
            [pltpu.SemaphoreType.DMA for _ in range(2)],
            [pltpu.SemaphoreType.DMA for _ in range(2)],
        ],
    )
    def gather_kernel(
        table_hbm, l_hbm, out_hbm, idx_v, rows_v, isem, gsem, osem
    ):
        wid = lax.axis_index("s") * _NC + lax.axis_index("c")
        win0 = wid * n_win

        def start_fetch(win, buf):
            base = (win0 + win) * wentries
            pltpu.async_copy(l_hbm.at[pl.ds(base, wentries)], idx_v[buf],
                             isem[buf]).wait()
            src = table_hbm.at[plsc.Indices(idx_v[buf], ignored_value=_SENT)]
            pltpu.make_async_copy(src, rows_v[buf], gsem[buf]).start()

        def finish(win, buf):
            src = table_hbm.at[plsc.Indices(idx_v[buf], ignored_value=_SENT)]
            pltpu.make_async_copy(src, rows_v[buf], gsem[buf]).wait()
            b0 = (win0 + win) * rows_per_win
            for r in range(rows_per_win):
                pltpu.make_async_copy(
                    rows_v[buf].at[pl.ds(r * hist, hist)],
                    out_hbm.at[b0 + r],
                    osem[buf],
                ).start()

        def drain_out(buf):
            for r in range(rows_per_win):
                pltpu.make_async_copy(
                    rows_v[buf].at[pl.ds(r * hist, hist)],
                    out_hbm.at[win0 * rows_per_win + r],
                    osem[buf],
                ).wait()

        start_fetch(0, 0)

        @pl.loop(0, n_win - 2, step=2)
        def _(w):
            start_fetch(w + 1, 1)
            finish(w, 0)
            drain_out(0)
            start_fetch(w + 2, 0)
            finish(w + 1, 1)
            drain_out(1)

        start_fetch(n_win - 1, 1)
        finish(n_win - 2, 0)
        drain_out(0)
        finish(n_win - 1, 1)
        drain_out(1)

    return gather_kernel(weight, idx_list)
